# baseline (speedup 1.0000x reference)
import jax
import jax.numpy as jnp
from jax import lax
from jax.experimental import pallas as pl
from jax.experimental.pallas import tpu as pltpu
from jax.experimental.pallas import tpu_sc as plsc


def _gather_body(nc, bpw, table_hbm, idx_hbm, out_hbm, idx_v, rows_v, sem):
    wid = lax.axis_index("s") * nc + lax.axis_index("c")
    base = wid * bpw
    pltpu.sync_copy(idx_hbm.at[pl.ds(base, bpw)], idx_v)
    pltpu.async_copy(table_hbm.at[idx_v], rows_v, sem).wait()
    pltpu.sync_copy(rows_v, out_hbm.at[pl.ds(base, bpw)])


def kernel(x, embeddings):
    b = x.shape[0]
    cap, d = embeddings.shape
    idx = jax.random.randint(jax.random.key(1), (b,), 0, cap, dtype=jnp.int32)

    info = plsc.get_sparse_core_info()
    nc, ns = info.num_cores, info.num_subcores
    nw = nc * ns
    bpw = b // nw

    gather = pl.kernel(
        lambda *refs: _gather_body(nc, bpw, *refs),
        mesh=plsc.VectorSubcoreMesh(core_axis_name="c", subcore_axis_name="s"),
        out_type=jax.ShapeDtypeStruct((b, d), jnp.float32),
        scratch_types=[
            pltpu.VMEM((bpw,), jnp.int32),
            pltpu.VMEM((bpw, d), jnp.float32),
            pltpu.SemaphoreType.DMA,
        ],
        compiler_params=pltpu.CompilerParams(use_tc_tiling_on_sc=False),
    )
    return gather(embeddings, idx)


# speedup vs baseline: 3.9139x; 3.9139x over previous
import jax
import jax.numpy as jnp
from jax import lax
from jax.experimental import pallas as pl
from jax.experimental.pallas import tpu as pltpu
from jax.experimental.pallas import tpu_sc as plsc


def _gather_body(nc, bpw, tab_hbm, idx_hbm, out_hbm, idx_v, block_v, stage_v, sem):
    wid = lax.axis_index("s") * nc + lax.axis_index("c")
    base = wid * bpw
    pltpu.sync_copy(idx_hbm.at[pl.ds(base, bpw)], idx_v)

    lane = lax.iota(jnp.int32, 16)

    def body(g, carry):
        chunk = idx_v[pl.ds(g * 16, 16)]
        for q in range(16):
            r = chunk[q]
            rj = pl.multiple_of((r // 128) * 128, 128)
            c = r % 128
            pltpu.async_copy(
                tab_hbm.at[pl.ds(0, 64), pl.ds(rj, 128)], block_v, sem
            ).wait()
            cvec = jnp.full((16,), c, jnp.int32)
            jvec = jnp.full((16,), g * 16 + q, jnp.int32)
            for p in range(4):
                dvec = lane + (p * 16)
                vals = plsc.load_gather(block_v, [dvec, cvec])
                plsc.store_scatter(stage_v, [dvec, jvec], vals)
        return carry

    lax.fori_loop(0, bpw // 16, body, 0, unroll=False)
    pltpu.sync_copy(stage_v, out_hbm.at[pl.ds(0, 64), pl.ds(base, bpw)])


def kernel(x, embeddings):
    b = x.shape[0]
    cap, d = embeddings.shape
    idx = jax.random.randint(jax.random.key(1), (b,), 0, cap, dtype=jnp.int32)

    info = plsc.get_sparse_core_info()
    nc, ns = info.num_cores, info.num_subcores
    nw = nc * ns
    bpw = b // nw

    gather = pl.kernel(
        lambda *refs: _gather_body(nc, bpw, *refs),
        mesh=plsc.VectorSubcoreMesh(core_axis_name="c", subcore_axis_name="s"),
        out_type=jax.ShapeDtypeStruct((d, b), jnp.float32),
        scratch_types=[
            pltpu.VMEM((bpw,), jnp.int32),
            pltpu.VMEM((d, 128), jnp.float32),
            pltpu.VMEM((d, bpw), jnp.float32),
            pltpu.SemaphoreType.DMA,
        ],
        compiler_params=pltpu.CompilerParams(needs_layout_passes=False),
    )
    out_t = gather(embeddings.T, idx)
    return out_t.T


# speedup vs baseline: 7.3359x; 1.8743x over previous
import jax
import jax.numpy as jnp
from jax import lax
from jax.experimental import pallas as pl
from jax.experimental.pallas import tpu as pltpu
from jax.experimental.pallas import tpu_sc as plsc

_NBUF = 4


def _gather_body(nc, bpw, tab_hbm, idx_hbm, out_hbm, idx_v,
                 b0, b1, b2, b3, stage_v, s0, s1, s2, s3):
    bufs = (b0, b1, b2, b3)
    sems = (s0, s1, s2, s3)
    wid = lax.axis_index("s") * nc + lax.axis_index("c")
    base = wid * bpw
    pltpu.sync_copy(idx_hbm.at[pl.ds(base, bpw)], idx_v)

    lane = lax.iota(jnp.int32, 16)

    def fire(r, buf, sem):
        rj = pl.multiple_of((r // 128) * 128, 128)
        return pltpu.async_copy(tab_hbm.at[pl.ds(0, 64), pl.ds(rj, 128)], buf, sem)

    def group(g, carry):
        chunk = idx_v[pl.ds(g * 16, 16)]
        cps = [fire(chunk[j], bufs[j], sems[j]) for j in range(_NBUF)]
        for j in range(16):
            bsel = j % _NBUF
            cps[j].wait()
            cvec = jnp.full((16,), chunk[j] % 128, jnp.int32)
            jvec = jnp.full((16,), g * 16 + j, jnp.int32)
            for p in range(4):
                dvec = lane + (p * 16)
                vals = plsc.load_gather(bufs[bsel], [dvec, cvec])
                plsc.store_scatter(stage_v, [dvec, jvec], vals)
            if j + _NBUF < 16:
                cps.append(fire(chunk[j + _NBUF], bufs[bsel], sems[bsel]))
        return carry

    lax.fori_loop(0, bpw // 16, group, 0, unroll=False)
    pltpu.sync_copy(stage_v, out_hbm.at[pl.ds(0, 64), pl.ds(base, bpw)])


def kernel(x, embeddings):
    b = x.shape[0]
    cap, d = embeddings.shape
    idx = jax.random.randint(jax.random.key(1), (b,), 0, cap, dtype=jnp.int32)

    info = plsc.get_sparse_core_info()
    nc, ns = info.num_cores, info.num_subcores
    nw = nc * ns
    bpw = b // nw

    gather = pl.kernel(
        lambda *refs: _gather_body(nc, bpw, *refs),
        mesh=plsc.VectorSubcoreMesh(core_axis_name="c", subcore_axis_name="s"),
        out_type=jax.ShapeDtypeStruct((d, b), jnp.float32),
        scratch_types=(
            [pltpu.VMEM((bpw,), jnp.int32)]
            + [pltpu.VMEM((d, 128), jnp.float32) for _ in range(_NBUF)]
            + [pltpu.VMEM((d, bpw), jnp.float32)]
            + [pltpu.SemaphoreType.DMA for _ in range(_NBUF)]
        ),
        compiler_params=pltpu.CompilerParams(needs_layout_passes=False),
    )
    out_t = gather(embeddings.T, idx)
    return out_t.T


# speedup vs baseline: 8.8813x; 1.2107x over previous
import jax
import jax.numpy as jnp
from jax import lax
from jax.experimental import pallas as pl
from jax.experimental.pallas import tpu as pltpu
from jax.experimental.pallas import tpu_sc as plsc

_NBUF = 8


def _gather_body(nc, bpw, tab_hbm, idx_hbm, out_hbm, idx_v,
                 b0, b1, b2, b3, b4, b5, b6, b7, stage_v,
                 s0, s1, s2, s3, s4, s5, s6, s7):
    bufs = (b0, b1, b2, b3, b4, b5, b6, b7)
    sems = (s0, s1, s2, s3, s4, s5, s6, s7)
    wid = lax.axis_index("s") * nc + lax.axis_index("c")
    base = wid * bpw
    ngroups = bpw // 16
    pltpu.sync_copy(idx_hbm.at[pl.ds(base, bpw)], idx_v)

    lane = lax.iota(jnp.int32, 16)
    pend = [[] for _ in range(_NBUF)]

    def fire(r, bsel):
        rj = pl.multiple_of((r // 128) * 128, 128)
        pend[bsel].append(
            pltpu.async_copy(
                tab_hbm.at[pl.ds(0, 64), pl.ds(rj, 128)], bufs[bsel], sems[bsel]
            )
        )

    chunk0 = idx_v[pl.ds(0, 16)]
    for j in range(_NBUF):
        fire(chunk0[j], j)

    def group(g, carry):
        chunk = idx_v[pl.ds(g * 16, 16)]
        noff = jnp.where(g + 1 < ngroups, (g + 1) * 16, 0)
        chunk_next = idx_v[pl.ds(noff, 16)]
        for j in range(16):
            bsel = j % _NBUF
            pend[bsel].pop(0).wait()
            cvec = jnp.full((16,), chunk[j] % 128, jnp.int32)
            jvec = jnp.full((16,), g * 16 + j, jnp.int32)
            for p in range(4):
                dvec = lane + (p * 16)
                vals = plsc.load_gather(bufs[bsel], [dvec, cvec])
                plsc.store_scatter(stage_v, [dvec, jvec], vals)
            if j < 16 - _NBUF:
                fire(chunk[j + _NBUF], bsel)
            else:
                @pl.when(g + 1 < ngroups)
                def _():
                    fire(chunk_next[j - (16 - _NBUF)], bsel)
        return carry

    lax.fori_loop(0, ngroups, group, 0, unroll=False)
    pltpu.sync_copy(stage_v, out_hbm.at[pl.ds(0, 64), pl.ds(base, bpw)])


def kernel(x, embeddings):
    b = x.shape[0]
    cap, d = embeddings.shape
    idx = jax.random.randint(jax.random.key(1), (b,), 0, cap, dtype=jnp.int32)

    info = plsc.get_sparse_core_info()
    nc, ns = info.num_cores, info.num_subcores
    nw = nc * ns
    bpw = b // nw

    gather = pl.kernel(
        lambda *refs: _gather_body(nc, bpw, *refs),
        mesh=plsc.VectorSubcoreMesh(core_axis_name="c", subcore_axis_name="s"),
        out_type=jax.ShapeDtypeStruct((d, b), jnp.float32),
        scratch_types=(
            [pltpu.VMEM((bpw,), jnp.int32)]
            + [pltpu.VMEM((d, 128), jnp.float32) for _ in range(_NBUF)]
            + [pltpu.VMEM((d, bpw), jnp.float32)]
            + [pltpu.SemaphoreType.DMA for _ in range(_NBUF)]
        ),
        compiler_params=pltpu.CompilerParams(needs_layout_passes=False),
    )
    out_t = gather(embeddings.T, idx)
    return out_t.T
